# baseline (speedup 1.0000x reference)
import jax
import jax.numpy as jnp
from jax import lax
from jax.experimental import pallas as pl
from jax.experimental.pallas import tpu as pltpu
from jax.experimental.pallas import tpu_sc as plsc

_NC = 2
_NS = 16
_NW = _NC * _NS
_CH = 256
_NB = 5
_K = 3


def _gather_body(x_hbm, table_hbm, out_hbm, idx_all, rows_v, sem_g, sem_w):
    n = out_hbm.shape[0]
    per_w = n // _NW
    nchunk = per_w // _CH
    wid = lax.axis_index("s") * _NC + lax.axis_index("c")
    base = wid * per_w

    def idx_slice(i):
        return idx_all.at[pl.ds(i * _CH, _CH)]

    def out_slice(i):
        return out_hbm.at[pl.ds(base + i * _CH, _CH)]

    def start_gather(i, b):
        pltpu.async_copy(table_hbm.at[idx_slice(i)], rows_v.at[b], sem_g.at[b])

    def wait_gather(i, b):
        pltpu.make_async_copy(
            table_hbm.at[idx_slice(i)], rows_v.at[b], sem_g.at[b]
        ).wait()

    def start_wb(i, b):
        pltpu.async_copy(rows_v.at[b], out_slice(i), sem_w.at[b])

    def wait_wb(i, b):
        pltpu.make_async_copy(rows_v.at[b], out_slice(i), sem_w.at[b]).wait()

    pltpu.sync_copy(x_hbm.at[pl.ds(base, per_w)], idx_all)

    def step(g, carry):
        for b in range(_NB):
            t = g * _NB + b

            @pl.when(t >= _NB)
            def _drain_slot():
                wait_wb(t - _NB, b)

            start_gather(t, b)

            c = t - _K
            cb = (b + _NB - _K) % _NB

            @pl.when(t >= _K)
            def _consume():
                wait_gather(c, cb)
                start_wb(c, cb)
        return carry

    lax.fori_loop(0, nchunk // _NB, step, 0)

    for j in range(nchunk - _K, nchunk):
        cb = j % _NB
        wait_gather(j, cb)
        start_wb(j, cb)
    for b in range(_NB):
        wait_wb(nchunk - _NB + b, b)


def kernel(x, table):
    b, h = x.shape
    n = b * h
    dim = table.shape[1]
    xf = x.reshape(n)
    mesh = plsc.VectorSubcoreMesh(core_axis_name="c", subcore_axis_name="s")
    out = pl.kernel(
        _gather_body,
        out_type=jax.ShapeDtypeStruct((n, dim), table.dtype),
        mesh=mesh,
        scratch_types=[
            pltpu.VMEM((n // _NW,), jnp.int32),
            pltpu.VMEM((_NB, _CH, dim), jnp.float32),
            pltpu.SemaphoreType.DMA((_NB,)),
            pltpu.SemaphoreType.DMA((_NB,)),
        ],
        compiler_params=pltpu.CompilerParams(use_tc_tiling_on_sc=False),
    )(xf, table)
    return out.reshape(b, h, dim)
